# baseline (speedup 1.0000x reference)
import functools
import math

import jax
import jax.numpy as jnp
from jax import lax
from jax.experimental import pallas as pl
from jax.experimental.pallas import tpu as pltpu

K = 16
R = 256
SUB = 4
C = 128


def _fused_kernel(x_ref, t_ref, c_ref, wmsg_ref, bmsg_ref, wtime_ref,
                  wctx_ref, wout_ref, bout_ref, out_ref, sq_ref):
    r = pl.program_id(1)
    xs = x_ref[0]
    n_nodes = xs.shape[0]

    @pl.when(r == 0)
    def _():
        sq_ref[...] = jnp.sum(xs * xs, axis=1)[None, :]

    half = wtime_ref.shape[0] // 2
    freq_i = jax.lax.broadcasted_iota(jnp.int32, (1, half), 1).astype(
        jnp.float32)
    freqs = jnp.exp(freq_i * (-math.log(10000.0) / half))
    args = t_ref[0] * freqs
    temb = jnp.concatenate([jnp.cos(args), jnp.sin(args)], axis=-1)
    cond = (jnp.dot(temb, wtime_ref[...], preferred_element_type=jnp.float32)
            + jnp.dot(c_ref[0], wctx_ref[...],
                      preferred_element_type=jnp.float32))

    def _cross(i):
        xr = x_ref[0, pl.ds((r * SUB + i) * R, R), :]
        return xr, jax.lax.dot_general(
            xr, xs, (((1,), (1,)), ((), ())),
            preferred_element_type=jnp.float32)

    ones_col = jnp.ones((n_nodes, 1), dtype=jnp.float32)
    nxt = _cross(0)
    for sub in range(SUB):
        rr = r * SUB + sub
        xr, cross = nxt
        if sub + 1 < SUB:
            nxt = _cross(sub + 1)
        s = sq_ref[...] - 2.0 * cross

        nchunk = n_nodes // C
        inf = jnp.full((R, C), jnp.inf, dtype=jnp.float32)
        m1, m2, m3, m4 = inf, inf, inf, inf
        for g in range(nchunk):
            v = s[:, g * C:(g + 1) * C]
            b1 = jnp.maximum(m1, v)
            m1 = jnp.minimum(m1, v)
            b2 = jnp.maximum(m2, b1)
            m2 = jnp.minimum(m2, b1)
            b3 = jnp.maximum(m3, b2)
            m3 = jnp.minimum(m3, b2)
            m4 = jnp.minimum(m4, b3)
        tau = None
        for _ in range(K + 1):
            tau = jnp.min(m1, axis=1, keepdims=True)
            eq = m1 == tau
            m1 = jnp.where(eq, m2, m1)
            m2 = jnp.where(eq, m3, m2)
            m3 = jnp.where(eq, m4, m3)
            m4 = jnp.where(eq, jnp.inf, m4)
        selmask = jnp.where(s <= tau, 1.0, 0.0)
        cnt = jnp.dot(selmask, ones_col,
                      preferred_element_type=jnp.float32)
        nbad = jnp.sum(jnp.where(cnt == float(K + 1), 0.0, 1.0))

        def _exact_mask(s=s):
            w = s
            m = None
            for _ in range(K + 1):
                m = jnp.min(w, axis=1, keepdims=True)
                w = jnp.where(w == m, jnp.inf, w)
            return jnp.where(s <= m, 1.0, 0.0)

        mask = lax.cond(nbad > 0.0, _exact_mask,
                        lambda selmask=selmask: selmask)

        nbr = jnp.dot(mask, xs,
                      preferred_element_type=jnp.float32) - xr
        agg = (jnp.dot(nbr, wmsg_ref[...], preferred_element_type=jnp.float32)
               + float(K) * bmsg_ref[...][None, :])
        h = jnp.maximum(xr + agg + cond, 0.0)
        out_ref[0, sub * R:(sub + 1) * R, :] = (
            jnp.dot(h, wout_ref[...], preferred_element_type=jnp.float32)
            + bout_ref[...][None, :])


def kernel(x, t, c_vector, W_msg, b_msg, W_time, W_ctx, W_out, b_out):
    B, N, D = x.shape
    CTX = c_vector.shape[1]
    nb = N // (R * SUB)
    grid = (B, nb)
    out = pl.pallas_call(
        _fused_kernel,
        grid=grid,
        in_specs=[
            pl.BlockSpec((1, N, D), lambda b, r: (b, 0, 0)),
            pl.BlockSpec((1, 1, 1), lambda b, r: (b, 0, 0)),
            pl.BlockSpec((1, 1, CTX), lambda b, r: (b, 0, 0)),
            pl.BlockSpec((D, D), lambda b, r: (0, 0)),
            pl.BlockSpec((D,), lambda b, r: (0,)),
            pl.BlockSpec((D, D), lambda b, r: (0, 0)),
            pl.BlockSpec((CTX, D), lambda b, r: (0, 0)),
            pl.BlockSpec((D, D), lambda b, r: (0, 0)),
            pl.BlockSpec((D,), lambda b, r: (0,)),
        ],
        out_specs=pl.BlockSpec((1, R * SUB, D), lambda b, r: (b, r, 0)),
        out_shape=jax.ShapeDtypeStruct((B, N, D), jnp.float32),
        scratch_shapes=[pltpu.VMEM((1, N), jnp.float32)],
    )(x, t.reshape(B, 1, 1), c_vector.reshape(B, 1, CTX), W_msg, b_msg,
      W_time, W_ctx, W_out, b_out)
    return out


# speedup vs baseline: 1.1171x; 1.1171x over previous
import functools
import math

import jax
import jax.numpy as jnp
from jax import lax
from jax.experimental import pallas as pl
from jax.experimental.pallas import tpu as pltpu

K = 16
R = 256
SUB = 4
C = 128


def _fused_kernel(x_ref, t_ref, c_ref, wmsg_ref, bmsg_ref, wtime_ref,
                  wctx_ref, wout_ref, bout_ref, out_ref, sq_ref):
    r = pl.program_id(1)
    xs = x_ref[0]
    n_nodes = xs.shape[0]

    @pl.when(r == 0)
    def _():
        sq_ref[...] = jnp.sum(xs * xs, axis=1)[None, :]

    half = wtime_ref.shape[0] // 2
    freq_i = jax.lax.broadcasted_iota(jnp.int32, (1, half), 1).astype(
        jnp.float32)
    freqs = jnp.exp(freq_i * (-math.log(10000.0) / half))
    args = t_ref[0] * freqs
    temb = jnp.concatenate([jnp.cos(args), jnp.sin(args)], axis=-1)
    cond = (jnp.dot(temb, wtime_ref[...], preferred_element_type=jnp.float32)
            + jnp.dot(c_ref[0], wctx_ref[...],
                      preferred_element_type=jnp.float32))

    def _cross(i):
        xr = x_ref[0, pl.ds((r * SUB + i) * R, R), :]
        return xr, jax.lax.dot_general(
            xr, xs, (((1,), (1,)), ((), ())),
            preferred_element_type=jnp.float32)

    nxt = _cross(0)
    for sub in range(SUB):
        rr = r * SUB + sub
        xr, cross = nxt
        if sub + 1 < SUB:
            nxt = _cross(sub + 1)
        s = sq_ref[...] - 2.0 * cross

        nchunk = n_nodes // C
        inf = jnp.full((R, C), jnp.inf, dtype=jnp.float32)
        m1, m2, m3, m4 = inf, inf, inf, inf
        for g in range(nchunk):
            v = s[:, g * C:(g + 1) * C]
            b1 = jnp.maximum(m1, v)
            m1 = jnp.minimum(m1, v)
            b2 = jnp.maximum(m2, b1)
            m2 = jnp.minimum(m2, b1)
            b3 = jnp.maximum(m3, b2)
            m3 = jnp.minimum(m3, b2)
            m4 = jnp.minimum(m4, b3)
        tau = None
        for _ in range(K + 1):
            tau = jnp.min(m1, axis=1, keepdims=True)
            eq = m1 == tau
            m1 = jnp.where(eq, m2, m1)
            m2 = jnp.where(eq, m3, m2)
            m3 = jnp.where(eq, m4, m3)
            m4 = jnp.where(eq, jnp.inf, m4)
        selmask = jnp.where(s <= tau, 1.0, 0.0)
        cnt = jnp.sum(selmask, axis=1, keepdims=True)
        nbad = jnp.sum(jnp.where(cnt == float(K + 1), 0.0, 1.0))

        def _exact_mask(s=s):
            w = s
            m = None
            for _ in range(K + 1):
                m = jnp.min(w, axis=1, keepdims=True)
                w = jnp.where(w == m, jnp.inf, w)
            return jnp.where(s <= m, 1.0, 0.0)

        mask = lax.cond(nbad > 0.0, _exact_mask,
                        lambda selmask=selmask: selmask)

        nbr = jnp.dot(mask, xs,
                      preferred_element_type=jnp.float32) - xr
        agg = (jnp.dot(nbr, wmsg_ref[...], preferred_element_type=jnp.float32)
               + float(K) * bmsg_ref[...][None, :])
        h = jnp.maximum(xr + agg + cond, 0.0)
        out_ref[0, sub * R:(sub + 1) * R, :] = (
            jnp.dot(h, wout_ref[...], preferred_element_type=jnp.float32)
            + bout_ref[...][None, :])


def kernel(x, t, c_vector, W_msg, b_msg, W_time, W_ctx, W_out, b_out):
    B, N, D = x.shape
    CTX = c_vector.shape[1]
    nb = N // (R * SUB)
    grid = (B, nb)
    out = pl.pallas_call(
        _fused_kernel,
        grid=grid,
        in_specs=[
            pl.BlockSpec((1, N, D), lambda b, r: (b, 0, 0)),
            pl.BlockSpec((1, 1, 1), lambda b, r: (b, 0, 0)),
            pl.BlockSpec((1, 1, CTX), lambda b, r: (b, 0, 0)),
            pl.BlockSpec((D, D), lambda b, r: (0, 0)),
            pl.BlockSpec((D,), lambda b, r: (0,)),
            pl.BlockSpec((D, D), lambda b, r: (0, 0)),
            pl.BlockSpec((CTX, D), lambda b, r: (0, 0)),
            pl.BlockSpec((D, D), lambda b, r: (0, 0)),
            pl.BlockSpec((D,), lambda b, r: (0,)),
        ],
        out_specs=pl.BlockSpec((1, R * SUB, D), lambda b, r: (b, r, 0)),
        out_shape=jax.ShapeDtypeStruct((B, N, D), jnp.float32),
        scratch_shapes=[pltpu.VMEM((1, N), jnp.float32)],
    )(x, t.reshape(B, 1, 1), c_vector.reshape(B, 1, CTX), W_msg, b_msg,
      W_time, W_ctx, W_out, b_out)
    return out


# speedup vs baseline: 1.2631x; 1.1307x over previous
import functools
import math

import jax
import jax.numpy as jnp
from jax import lax
from jax.experimental import pallas as pl
from jax.experimental.pallas import tpu as pltpu

K = 16
R = 512
SUB = 2
C = 128


def _fused_kernel(x_ref, t_ref, c_ref, wmsg_ref, bmsg_ref, wtime_ref,
                  wctx_ref, wout_ref, bout_ref, out_ref, sq_ref):
    r = pl.program_id(1)
    xs = x_ref[0]
    n_nodes = xs.shape[0]

    @pl.when(r == 0)
    def _():
        sq_ref[...] = jnp.sum(xs * xs, axis=1)[None, :]

    half = wtime_ref.shape[0] // 2
    freq_i = jax.lax.broadcasted_iota(jnp.int32, (1, half), 1).astype(
        jnp.float32)
    freqs = jnp.exp(freq_i * (-math.log(10000.0) / half))
    args = t_ref[0] * freqs
    temb = jnp.concatenate([jnp.cos(args), jnp.sin(args)], axis=-1)
    cond = (jnp.dot(temb, wtime_ref[...], preferred_element_type=jnp.float32)
            + jnp.dot(c_ref[0], wctx_ref[...],
                      preferred_element_type=jnp.float32))

    def _cross(i):
        xr = x_ref[0, pl.ds((r * SUB + i) * R, R), :]
        return xr, jax.lax.dot_general(
            xr, xs, (((1,), (1,)), ((), ())),
            preferred_element_type=jnp.float32)

    nxt = _cross(0)
    for sub in range(SUB):
        rr = r * SUB + sub
        xr, cross = nxt
        if sub + 1 < SUB:
            nxt = _cross(sub + 1)
        s = sq_ref[...] - 2.0 * cross

        nchunk = n_nodes // C
        inf = jnp.full((R, C), jnp.inf, dtype=jnp.float32)
        m1, m2, m3, m4 = inf, inf, inf, inf
        for g in range(nchunk):
            v = s[:, g * C:(g + 1) * C]
            b1 = jnp.maximum(m1, v)
            m1 = jnp.minimum(m1, v)
            b2 = jnp.maximum(m2, b1)
            m2 = jnp.minimum(m2, b1)
            b3 = jnp.maximum(m3, b2)
            m3 = jnp.minimum(m3, b2)
            m4 = jnp.minimum(m4, b3)
        tau = None
        for _ in range(K + 1):
            tau = jnp.min(m1, axis=1, keepdims=True)
            eq = m1 == tau
            m1 = jnp.where(eq, m2, m1)
            m2 = jnp.where(eq, m3, m2)
            m3 = jnp.where(eq, m4, m3)
            m4 = jnp.where(eq, jnp.inf, m4)
        selmask = jnp.where(s <= tau, 1.0, 0.0)
        cnt = jnp.sum(selmask, axis=1, keepdims=True)
        nbad = jnp.sum(jnp.where(cnt == float(K + 1), 0.0, 1.0))

        def _exact_mask(s=s):
            w = s
            m = None
            for _ in range(K + 1):
                m = jnp.min(w, axis=1, keepdims=True)
                w = jnp.where(w == m, jnp.inf, w)
            return jnp.where(s <= m, 1.0, 0.0)

        mask = lax.cond(nbad > 0.0, _exact_mask,
                        lambda selmask=selmask: selmask)

        nbr = jnp.dot(mask, xs,
                      preferred_element_type=jnp.float32) - xr
        agg = (jnp.dot(nbr, wmsg_ref[...], preferred_element_type=jnp.float32)
               + float(K) * bmsg_ref[...][None, :])
        h = jnp.maximum(xr + agg + cond, 0.0)
        out_ref[0, sub * R:(sub + 1) * R, :] = (
            jnp.dot(h, wout_ref[...], preferred_element_type=jnp.float32)
            + bout_ref[...][None, :])


def kernel(x, t, c_vector, W_msg, b_msg, W_time, W_ctx, W_out, b_out):
    B, N, D = x.shape
    CTX = c_vector.shape[1]
    nb = N // (R * SUB)
    grid = (B, nb)
    out = pl.pallas_call(
        _fused_kernel,
        grid=grid,
        in_specs=[
            pl.BlockSpec((1, N, D), lambda b, r: (b, 0, 0)),
            pl.BlockSpec((1, 1, 1), lambda b, r: (b, 0, 0)),
            pl.BlockSpec((1, 1, CTX), lambda b, r: (b, 0, 0)),
            pl.BlockSpec((D, D), lambda b, r: (0, 0)),
            pl.BlockSpec((D,), lambda b, r: (0,)),
            pl.BlockSpec((D, D), lambda b, r: (0, 0)),
            pl.BlockSpec((CTX, D), lambda b, r: (0, 0)),
            pl.BlockSpec((D, D), lambda b, r: (0, 0)),
            pl.BlockSpec((D,), lambda b, r: (0,)),
        ],
        out_specs=pl.BlockSpec((1, R * SUB, D), lambda b, r: (b, r, 0)),
        out_shape=jax.ShapeDtypeStruct((B, N, D), jnp.float32),
        scratch_shapes=[pltpu.VMEM((1, N), jnp.float32)],
    )(x, t.reshape(B, 1, 1), c_vector.reshape(B, 1, CTX), W_msg, b_msg,
      W_time, W_ctx, W_out, b_out)
    return out
